# Initial kernel scaffold; baseline (speedup 1.0000x reference)
import jax
import jax.numpy as jnp
from jax.experimental import pallas as pl


def kernel(k_new, v_new, k_cache, v_cache):
    raise NotImplementedError("write your pallas kernel here")


# speedup vs baseline: 45.3257x; 45.3257x over previous
import jax
import jax.numpy as jnp
from jax.experimental import pallas as pl

MAX_BATCH = 32
N_KV_HEADS = 8
Q_LEN = 16
HEAD_DIM = 128

_BLOCK_B = 4


def _copy_body(k_ref, v_ref, ok_ref, ov_ref):
    ok_ref[...] = k_ref[...]
    ov_ref[...] = v_ref[...]


def kernel(k_new, v_new, k_cache, v_cache):
    del k_cache, v_cache
    shape = jax.ShapeDtypeStruct(k_new.shape, k_new.dtype)
    spec = pl.BlockSpec(
        (_BLOCK_B, N_KV_HEADS, Q_LEN, HEAD_DIM), lambda i: (i, 0, 0, 0)
    )
    out_k, out_v = pl.pallas_call(
        _copy_body,
        grid=(MAX_BATCH // _BLOCK_B,),
        in_specs=[spec, spec],
        out_specs=[spec, spec],
        out_shape=[shape, shape],
    )(k_new, v_new)
    return (out_k, out_v)
